# baseline (speedup 1.0000x reference)
import jax
import jax.numpy as jnp
from jax.experimental import pallas as pl

BM = 128


def _fused_body(p_ref, x_ref, w1_ref, b1_ref, w2_ref, b2_ref, out_ref):
    xb = x_ref[...]
    resid = xb - xb.astype(jnp.bfloat16).astype(jnp.float32)
    pooled = (jnp.dot(p_ref[...], xb, preferred_element_type=jnp.float32)
              + jnp.dot(p_ref[...], resid,
                        preferred_element_type=jnp.float32))
    h = jnp.dot(pooled, w1_ref[...], preferred_element_type=jnp.float32)
    h = h + b1_ref[...]
    h = 0.5 * h * (1.0 + jax.lax.erf(h * 0.7071067811865476))
    out = jnp.dot(h, w2_ref[...], preferred_element_type=jnp.float32)
    out_ref[...] = out + b2_ref[...]


def kernel(x, W1, b1, W2, b2):
    n, c, d = x.shape
    h_dim = W1.shape[1]
    q = 256
    win = c // q
    m = n * q
    xf = x.reshape(m * win, d)
    rows = jax.lax.broadcasted_iota(jnp.int32, (BM, win * BM), 0)
    cols = jax.lax.broadcasted_iota(jnp.int32, (BM, win * BM), 1)
    pool_mat = jnp.where(cols // win == rows, 1.0 / win, 0.0).astype(jnp.float32)
    grid = (m // BM,)
    out = pl.pallas_call(
        _fused_body,
        grid=grid,
        in_specs=[
            pl.BlockSpec((BM, win * BM), lambda i: (0, 0)),
            pl.BlockSpec((BM * win, d), lambda i: (i, 0)),
            pl.BlockSpec((d, h_dim), lambda i: (0, 0)),
            pl.BlockSpec((1, h_dim), lambda i: (0, 0)),
            pl.BlockSpec((h_dim, d), lambda i: (0, 0)),
            pl.BlockSpec((1, d), lambda i: (0, 0)),
        ],
        out_specs=pl.BlockSpec((BM, d), lambda i: (i, 0)),
        out_shape=jax.ShapeDtypeStruct((m, d), jnp.float32),
    )(pool_mat, xf, W1, b1.reshape(1, h_dim), W2, b2.reshape(1, d))
    return out


# speedup vs baseline: 1.0658x; 1.0658x over previous
import jax
import jax.numpy as jnp
from jax.experimental import pallas as pl

BM = 128


def _fused_body(p_ref, xl_ref, xr_ref, w1_ref, b1_ref, w2_ref, b2_ref,
                out_ref):
    dh = xl_ref.shape[1]
    pooled_l = jnp.dot(p_ref[...], xl_ref[...],
                       preferred_element_type=jnp.float32)
    pooled_r = jnp.dot(p_ref[...], xr_ref[...],
                       preferred_element_type=jnp.float32)
    h = (jnp.dot(pooled_l, w1_ref[:dh, :],
                 preferred_element_type=jnp.float32)
         + jnp.dot(pooled_r, w1_ref[dh:, :],
                   preferred_element_type=jnp.float32))
    h = h + b1_ref[...]
    h = 0.5 * h * (1.0 + jax.lax.erf(h * 0.7071067811865476))
    out = jnp.dot(h, w2_ref[...], preferred_element_type=jnp.float32)
    out_ref[...] = out + b2_ref[...]


def kernel(x, W1, b1, W2, b2):
    n, c, d = x.shape
    h_dim = W1.shape[1]
    q = 256
    win = c // q
    m = n * q
    xf = x.reshape(m * win, d)
    rows = jax.lax.broadcasted_iota(jnp.int32, (BM, win * BM), 0)
    cols = jax.lax.broadcasted_iota(jnp.int32, (BM, win * BM), 1)
    pool_mat = jnp.where(cols // win == rows, 1.0 / win, 0.0).astype(jnp.float32)
    grid = (m // BM,)
    out = pl.pallas_call(
        _fused_body,
        grid=grid,
        in_specs=[
            pl.BlockSpec((BM, win * BM), lambda i: (0, 0)),
            pl.BlockSpec((BM * win, d // 2), lambda i: (i, 0)),
            pl.BlockSpec((BM * win, d // 2), lambda i: (i, 1)),
            pl.BlockSpec((d, h_dim), lambda i: (0, 0)),
            pl.BlockSpec((1, h_dim), lambda i: (0, 0)),
            pl.BlockSpec((h_dim, d), lambda i: (0, 0)),
            pl.BlockSpec((1, d), lambda i: (0, 0)),
        ],
        out_specs=pl.BlockSpec((BM, d), lambda i: (i, 0)),
        out_shape=jax.ShapeDtypeStruct((m, d), jnp.float32),
    )(pool_mat, xf, xf, W1, b1.reshape(1, h_dim), W2, b2.reshape(1, d))
    return out


# speedup vs baseline: 1.1004x; 1.0325x over previous
import jax
import jax.numpy as jnp
from jax.experimental import pallas as pl

BM = 128


def _fused_body(p_ref, x_ref, w1_ref, b1_ref, w2_ref, b2_ref, out_ref):
    pooled = jnp.dot(p_ref[...], x_ref[...],
                     preferred_element_type=jnp.float32)
    h = jnp.dot(pooled, w1_ref[...], preferred_element_type=jnp.float32)
    h = h + b1_ref[...]
    h = 0.5 * h * (1.0 + jax.lax.erf(h * 0.7071067811865476))
    out = jnp.dot(h, w2_ref[...], preferred_element_type=jnp.float32)
    out_ref[...] = out + b2_ref[...]


def kernel(x, W1, b1, W2, b2):
    n, c, d = x.shape
    h_dim = W1.shape[1]
    q = 256
    win = c // q
    m = n * q
    xf = x.reshape(m * win, d)
    rows = jax.lax.broadcasted_iota(jnp.int32, (BM, win * BM), 0)
    cols = jax.lax.broadcasted_iota(jnp.int32, (BM, win * BM), 1)
    pool_mat = jnp.where(cols // win == rows, 1.0 / win, 0.0).astype(jnp.float32)
    grid = (m // BM,)
    out = pl.pallas_call(
        _fused_body,
        grid=grid,
        in_specs=[
            pl.BlockSpec((BM, win * BM), lambda i: (0, 0)),
            pl.BlockSpec((BM * win, d), lambda i: (i, 0)),
            pl.BlockSpec((d, h_dim), lambda i: (0, 0)),
            pl.BlockSpec((1, h_dim), lambda i: (0, 0)),
            pl.BlockSpec((h_dim, d), lambda i: (0, 0)),
            pl.BlockSpec((1, d), lambda i: (0, 0)),
        ],
        out_specs=pl.BlockSpec((BM, d), lambda i: (i, 0)),
        out_shape=jax.ShapeDtypeStruct((m, d), jnp.float32),
    )(pool_mat, xf, W1, b1.reshape(1, h_dim), W2, b2.reshape(1, d))
    return out
